# speedup vs baseline: 283.0782x; 283.0782x over previous
import functools

import jax
import jax.numpy as jnp
from jax.experimental import pallas as pl


def _mlp_body(nk, nt, d_ref, v_ref, w1a_ref, w1bl_ref, w2_ref, b1_ref,
              b2_ref, o_ref):
    v = v_ref[...]
    seen = jnp.zeros(v.shape, jnp.bool_)
    for d in range(1, nk):
        shifted = jnp.concatenate(
            [jnp.full((d, nt), -1, v.dtype), v[: nk - d, :]], axis=0)
        seen = seen | (v == shifted)
    is_new = jnp.where(seen | (v == 0), 0.0, 1.0)

    a = jnp.dot(w1a_ref[...], d_ref[...], preferred_element_type=jnp.float32)
    b = jnp.dot(w1bl_ref[...], is_new, preferred_element_type=jnp.float32)
    h = jnp.tanh(a + b + b1_ref[...])
    logits = jnp.dot(w2_ref[...], h,
                     preferred_element_type=jnp.float32) + b2_ref[...]
    m = jnp.max(logits, axis=0, keepdims=True)
    e = jnp.exp(logits - m)
    o_ref[...] = e / jnp.sum(e, axis=0, keepdims=True)


def kernel(distances, values, W1, b1, W2, b2):
    B, S, K = distances.shape
    T = B * S
    HID = W1.shape[1]
    OUT = W2.shape[1]
    OUTP = 8

    dT = distances.reshape(T, K).T
    vT = values.astype(jnp.int32).reshape(T, K).T

    w1aT = W1[:K].T
    L = jnp.tril(jnp.ones((K, K), jnp.float32))
    w1blT = W1[K:].T @ L
    w2T = jnp.zeros((OUTP, HID), jnp.float32).at[:OUT].set(W2.T)
    b1c = b1.reshape(HID, 1)
    b2c = jnp.full((OUTP, 1), -1e9, jnp.float32).at[:OUT, 0].set(b2)

    out = pl.pallas_call(
        functools.partial(_mlp_body, K, T),
        out_shape=jax.ShapeDtypeStruct((OUTP, T), jnp.float32),
    )(dT, vT, w1aT, w1blT, w2T, b1c, b2c)

    return out[:OUT].T.reshape(B, S, OUT)
